# Initial kernel scaffold; baseline (speedup 1.0000x reference)
import jax
import jax.numpy as jnp
from jax.experimental import pallas as pl


def kernel(lv, gamma, beta, weight, bias, neighbor_idx):
    raise NotImplementedError("write your pallas kernel here")


# speedup vs baseline: 2.2024x; 2.2024x over previous
import functools

import jax
import jax.numpy as jnp
import numpy as np
from jax import lax
from jax.experimental import pallas as pl
from jax.experimental.pallas import tpu as pltpu
from jax.experimental.pallas import tpu_sc as plsc

N = 50000
C = 128
FE = 9
G = 32
EPS = 1e-5

NC = 2
NS = 16
NW = NC * NS
L = 16

NP = 51200
RW = NP // NW
BLK = 64
NB = RW // BLK

_RBLK = 2000
_RGRID = N // _RBLK


def _stats_body(lv_ref, sum_ref, sq_ref):
    i = pl.program_id(0)

    @pl.when(i == 0)
    def _():
        sum_ref[...] = jnp.zeros_like(sum_ref)
        sq_ref[...] = jnp.zeros_like(sq_ref)

    blk = lv_ref[...]
    sum_ref[...] += jnp.sum(blk, axis=0, keepdims=True)
    sq_ref[...] += jnp.sum(blk * blk, axis=0, keepdims=True)


def _stats(lv):
    return pl.pallas_call(
        _stats_body,
        grid=(_RGRID,),
        in_specs=[pl.BlockSpec((_RBLK, C), lambda i: (i, 0))],
        out_specs=[
            pl.BlockSpec((1, C), lambda i: (0, 0)),
            pl.BlockSpec((1, C), lambda i: (0, 0)),
        ],
        out_shape=[
            jax.ShapeDtypeStruct((1, C), jnp.float32),
            jax.ShapeDtypeStruct((1, C), jnp.float32),
        ],
    )(lv)


_GM = np.kron(np.eye(G, dtype=np.float32),
              np.ones((C // G, C // G), dtype=np.float32)) / float(N * (C // G))


def _norm_body(lv_ref, sum_ref, sq_ref, gamma_ref, beta_ref, gm_ref, x_ref):
    gm = gm_ref[...]
    mean = jnp.dot(sum_ref[...], gm, preferred_element_type=jnp.float32)
    esq = jnp.dot(sq_ref[...], gm, preferred_element_type=jnp.float32)
    var = esq - mean * mean
    inv = lax.rsqrt(var + EPS)
    a = gamma_ref[...] * inv
    b = beta_ref[...] - mean * a
    x_ref[...] = jnp.maximum(lv_ref[...] * a + b, 0.0)


def _normalize(lv, sums, sq, gamma, beta):
    gm = jnp.asarray(_GM)
    return pl.pallas_call(
        _norm_body,
        grid=(_RGRID,),
        in_specs=[
            pl.BlockSpec((_RBLK, C), lambda i: (i, 0)),
            pl.BlockSpec((1, C), lambda i: (0, 0)),
            pl.BlockSpec((1, C), lambda i: (0, 0)),
            pl.BlockSpec((1, C), lambda i: (0, 0)),
            pl.BlockSpec((1, C), lambda i: (0, 0)),
            pl.BlockSpec((C, C), lambda i: (0, 0)),
        ],
        out_specs=pl.BlockSpec((_RBLK, C), lambda i: (i, 0)),
        out_shape=jax.ShapeDtypeStruct((N, C), jnp.float32),
    )(lv, sums, sq, gamma.reshape(1, C), beta.reshape(1, C), gm)


def _sc_body(x_hbm, idxt_hbm, w_hbm, b_hbm, out_hbm,
             idx_v, taps_v, out_v, w_v, bias_v, gsem):
    wid = lax.axis_index("s") * NC + lax.axis_index("c")
    base0 = wid * RW
    pltpu.sync_copy(w_hbm, w_v)
    pltpu.sync_copy(b_hbm, bias_v)
    for f in range(FE):
        pltpu.sync_copy(idxt_hbm.at[pl.ds(f * NP + base0, RW)],
                        idx_v.at[pl.ds(f * RW, RW)])

    def blk(bi, carry):
        base = base0 + bi * BLK
        copies = [
            pltpu.async_copy(
                x_hbm.at[idx_v.at[pl.ds(f * RW + bi * BLK, BLK)]],
                taps_v.at[f], gsem)
            for f in range(FE)
        ]
        for cp in copies:
            cp.wait()
        for h in range(C // L):
            cs = h * L
            bias_h = bias_v[pl.ds(cs, L)]
            w_h = [w_v[f, pl.ds(cs, L)] for f in range(FE)]

            def row(r, c):
                acc = bias_h
                for f in range(FE):
                    acc = acc + taps_v[f, r, pl.ds(cs, L)] * w_h[f]
                out_v[r, pl.ds(cs, L)] = acc
                return c

            lax.fori_loop(0, BLK, row, 0, unroll=2)
        pltpu.sync_copy(out_v, out_hbm.at[pl.ds(base, BLK)])
        return carry

    lax.fori_loop(0, NB, blk, 0)


def _sc_conv(x, idxt, weight, bias):
    mesh = plsc.VectorSubcoreMesh(core_axis_name="c", subcore_axis_name="s")
    f = pl.kernel(
        _sc_body,
        out_type=jax.ShapeDtypeStruct((NP, C), jnp.float32),
        mesh=mesh,
        scratch_types=[
            pltpu.VMEM((FE * RW,), jnp.int32),
            pltpu.VMEM((FE, BLK, C), jnp.float32),
            pltpu.VMEM((BLK, C), jnp.float32),
            pltpu.VMEM((FE, C), jnp.float32),
            pltpu.VMEM((C,), jnp.float32),
            pltpu.SemaphoreType.DMA,
        ],
    )
    return f(x, idxt, weight, bias)


def kernel(lv, gamma, beta, weight, bias, neighbor_idx):
    sums, sq = _stats(lv)
    x = _normalize(lv, sums, sq, gamma, beta)
    idxt = jnp.zeros((FE, NP), jnp.int32).at[:, :N].set(neighbor_idx.T).reshape(-1)
    out = _sc_conv(x, idxt, weight, bias)
    return out[:N]


# speedup vs baseline: 3.1762x; 1.4422x over previous
import functools

import jax
import jax.numpy as jnp
import numpy as np
from jax import lax
from jax.experimental import pallas as pl
from jax.experimental.pallas import tpu as pltpu
from jax.experimental.pallas import tpu_sc as plsc

N = 50000
C = 128
FE = 9
G = 32
EPS = 1e-5

NC = 2
NS = 16
HC = C // 2

TB = 50048
NP = 51200
RWS = NP // NS
BLK = 40
NBLK = RWS // BLK
IBLK = FE * BLK

_RBLK = 2000
_RGRID = N // _RBLK


def _stats_body(lv_ref, sum_ref, sq_ref):
    i = pl.program_id(0)

    @pl.when(i == 0)
    def _():
        sum_ref[...] = jnp.zeros_like(sum_ref)
        sq_ref[...] = jnp.zeros_like(sq_ref)

    blk = lv_ref[...]
    sum_ref[...] += jnp.sum(blk, axis=0, keepdims=True)
    sq_ref[...] += jnp.sum(blk * blk, axis=0, keepdims=True)


def _stats(lv):
    return pl.pallas_call(
        _stats_body,
        grid=(_RGRID,),
        in_specs=[pl.BlockSpec((_RBLK, C), lambda i: (i, 0))],
        out_specs=[
            pl.BlockSpec((1, C), lambda i: (0, 0)),
            pl.BlockSpec((1, C), lambda i: (0, 0)),
        ],
        out_shape=[
            jax.ShapeDtypeStruct((1, C), jnp.float32),
            jax.ShapeDtypeStruct((1, C), jnp.float32),
        ],
    )(lv)


_GM = np.kron(np.eye(G, dtype=np.float32),
              np.ones((C // G, C // G), dtype=np.float32)) / float(N * (C // G))


def _norm_body(lv_ref, sum_ref, sq_ref, gamma_ref, beta_ref, gm_ref,
               x0_ref, x1_ref):
    gm = gm_ref[...]
    mean = jnp.dot(sum_ref[...], gm, preferred_element_type=jnp.float32)
    esq = jnp.dot(sq_ref[...], gm, preferred_element_type=jnp.float32)
    var = esq - mean * mean
    inv = lax.rsqrt(var + EPS)
    a = gamma_ref[...] * inv
    bb = beta_ref[...] - mean * a
    xb = jnp.maximum(lv_ref[...] * a + bb, 0.0).astype(jnp.bfloat16)
    x0_ref[...] = xb[:, :HC]
    x1_ref[...] = xb[:, HC:]


def _normalize(lv, sums, sq, gamma, beta):
    gm = jnp.asarray(_GM)
    return pl.pallas_call(
        _norm_body,
        grid=(_RGRID,),
        in_specs=[
            pl.BlockSpec((_RBLK, C), lambda i: (i, 0)),
            pl.BlockSpec((1, C), lambda i: (0, 0)),
            pl.BlockSpec((1, C), lambda i: (0, 0)),
            pl.BlockSpec((1, C), lambda i: (0, 0)),
            pl.BlockSpec((1, C), lambda i: (0, 0)),
            pl.BlockSpec((C, C), lambda i: (0, 0)),
        ],
        out_specs=[
            pl.BlockSpec((_RBLK, HC), lambda i: (i, 0)),
            pl.BlockSpec((_RBLK, HC), lambda i: (i, 0)),
        ],
        out_shape=[
            jax.ShapeDtypeStruct((TB, HC), jnp.bfloat16),
            jax.ShapeDtypeStruct((TB, HC), jnp.bfloat16),
        ],
    )(lv, sums, sq, gamma.reshape(1, C), beta.reshape(1, C), gm)


def _sc_body(xt0_hbm, xt1_hbm, idx_hbm, w_hbm, part_hbm,
             tab_s, idx_a, idx_b, taps_a, taps_b, out_a, out_b, w_v,
             gsa, gsb, isa, isb, osem):
    c = lax.axis_index("c")
    s = lax.axis_index("s")
    tpt = TB // NS

    @pl.when(c == 0)
    def _():
        pltpu.sync_copy(xt0_hbm.at[pl.ds(s * tpt, tpt)],
                        tab_s.at[pl.ds(s * tpt, tpt)])

    @pl.when(c == 1)
    def _():
        pltpu.sync_copy(xt1_hbm.at[pl.ds(s * tpt, tpt)],
                        tab_s.at[pl.ds(s * tpt, tpt)])

    pltpu.sync_copy(w_hbm.at[c], w_v)
    plsc.subcore_barrier()

    idx_refs = (idx_a, idx_b)
    taps_refs = (taps_a, taps_b)
    out_refs = (out_a, out_b)
    gsems = (gsa, gsb)
    isems = (isa, isb)
    gblk0 = s * NBLK

    def i_copy(bi, slot):
        return pltpu.make_async_copy(
            idx_hbm.at[pl.ds((gblk0 + bi) * IBLK, IBLK)],
            idx_refs[slot], isems[slot])

    def g_copy(slot, f):
        return pltpu.make_async_copy(
            tab_s.at[idx_refs[slot].at[pl.ds(f * BLK, BLK)]],
            taps_refs[slot].at[f], gsems[slot])

    def issue_g(slot):
        for f in range(FE):
            g_copy(slot, f).start()

    def wait_g(slot):
        for f in range(FE):
            g_copy(slot, f).wait()

    def out_copy(bi, slot):
        return pltpu.make_async_copy(
            out_refs[slot],
            part_hbm.at[c, pl.ds(s * RWS + bi * BLK, BLK)], osem)

    wr = [[w_v[f, pl.ds(j * 32, 32)] for j in range(HC // 32)]
          for f in range(FE)]

    def compute(slot):
        taps = taps_refs[slot]
        out_r = out_refs[slot]

        def rowfn(r, carry):
            for j in range(HC // 32):
                ps = [plsc.bitcast(taps[f, r, pl.ds(j * 16, 16)],
                                   jnp.bfloat16) * wr[f][j]
                      for f in range(FE)]
                while len(ps) > 1:
                    nxt = [ps[k] + ps[k + 1] for k in range(0, len(ps) - 1, 2)]
                    if len(ps) % 2:
                        nxt.append(ps[-1])
                    ps = nxt
                out_r[r, pl.ds(j * 32, 32)] = ps[0]
            return carry

        lax.fori_loop(0, BLK, rowfn, 0)

    i_copy(0, 0).start()
    i_copy(1, 1).start()
    i_copy(0, 0).wait()
    issue_g(0)
    i_copy(1, 1).wait()
    issue_g(1)

    def body2(i2, carry):
        b0 = i2 * 2
        b1 = b0 + 1

        wait_g(0)

        @pl.when(b0 + 2 < NBLK)
        def _():
            i_copy(b0 + 2, 0).start()

        @pl.when(b0 >= 2)
        def _():
            out_copy(b0 - 2, 0).wait()

        compute(0)
        out_copy(b0, 0).start()

        @pl.when(b0 + 2 < NBLK)
        def _():
            i_copy(b0 + 2, 0).wait()
            issue_g(0)

        wait_g(1)

        @pl.when(b1 + 2 < NBLK)
        def _():
            i_copy(b1 + 2, 1).start()

        @pl.when(b1 >= 2)
        def _():
            out_copy(b1 - 2, 1).wait()

        compute(1)
        out_copy(b1, 1).start()

        @pl.when(b1 + 2 < NBLK)
        def _():
            i_copy(b1 + 2, 1).wait()
            issue_g(1)

        return carry

    lax.fori_loop(0, NBLK // 2, body2, 0)
    out_copy(NBLK - 2, 0).wait()
    out_copy(NBLK - 1, 1).wait()


def _sc_conv(xt0, xt1, idx_blocks, w2):
    mesh = plsc.VectorSubcoreMesh(core_axis_name="c", subcore_axis_name="s")
    f = pl.kernel(
        _sc_body,
        out_type=jax.ShapeDtypeStruct((NC, NP, HC), jnp.bfloat16),
        mesh=mesh,
        compiler_params=pltpu.CompilerParams(
            needs_layout_passes=False, use_tc_tiling_on_sc=False),
        scratch_types=[
            pltpu.VMEM_SHARED((TB, HC // 2), jnp.int32),
            pltpu.VMEM((IBLK,), jnp.int32),
            pltpu.VMEM((IBLK,), jnp.int32),
            pltpu.VMEM((FE, BLK, HC // 2), jnp.int32),
            pltpu.VMEM((FE, BLK, HC // 2), jnp.int32),
            pltpu.VMEM((BLK, HC), jnp.bfloat16),
            pltpu.VMEM((BLK, HC), jnp.bfloat16),
            pltpu.VMEM((FE, HC), jnp.bfloat16),
            pltpu.SemaphoreType.DMA,
            pltpu.SemaphoreType.DMA,
            pltpu.SemaphoreType.DMA,
            pltpu.SemaphoreType.DMA,
            pltpu.SemaphoreType.DMA,
        ],
    )
    return f(xt0, xt1, idx_blocks, w2)


_CBLK = 2000


def _comb_body(p_ref, bias_ref, o_ref):
    p = p_ref[...]
    o_ref[...] = (jnp.concatenate([p[0], p[1]], axis=1).astype(jnp.float32)
                  + bias_ref[...])


def _combine(part, bias):
    return pl.pallas_call(
        _comb_body,
        grid=(N // _CBLK,),
        in_specs=[
            pl.BlockSpec((2, _CBLK, HC), lambda i: (0, i, 0)),
            pl.BlockSpec((1, C), lambda i: (0, 0)),
        ],
        out_specs=pl.BlockSpec((_CBLK, C), lambda i: (i, 0)),
        out_shape=jax.ShapeDtypeStruct((N, C), jnp.float32),
    )(part, bias.reshape(1, C))


def kernel(lv, gamma, beta, weight, bias, neighbor_idx):
    sums, sq = _stats(lv)
    x0, x1 = _normalize(lv, sums, sq, gamma, beta)
    xt0 = lax.bitcast_convert_type(x0.reshape(TB, HC // 2, 2), jnp.int32)
    xt1 = lax.bitcast_convert_type(x1.reshape(TB, HC // 2, 2), jnp.int32)

    t = neighbor_idx.T
    pos_p = lax.broadcasted_iota(jnp.int32, (FE, NP - N), 1)
    tpad = jnp.concatenate([t, pos_p & 4095], axis=1)
    idx_blocks = tpad.reshape(FE, NP // BLK, BLK).transpose(1, 0, 2).reshape(-1)

    w2 = weight.astype(jnp.bfloat16).reshape(FE, 2, HC).transpose(1, 0, 2)
    part = _sc_conv(xt0, xt1, idx_blocks, w2)
    return _combine(part, bias)
